# baseline (speedup 1.0000x reference)
import functools

import jax
import jax.numpy as jnp
from jax import lax
from jax.experimental import pallas as pl
from jax.experimental.pallas import tpu as pltpu
from jax.experimental.pallas import tpu_sc as plsc

CAP = 0.25
_LANES = 16


def _scores_body(x_ref, w_ref, b_ref, o_ref):
    xb = x_ref[0].astype(jnp.bfloat16).astype(jnp.float32)
    wb = w_ref[...].astype(jnp.bfloat16).astype(jnp.float32)
    tt = xb.shape[0]
    o_ref[...] = (jnp.sum(xb * wb, axis=-1) + b_ref[0]).reshape(1, 1, tt)


def _compute_scores(x, W, b, t1):
    B, T, D = x.shape
    TT = 1024
    nj = t1 // TT
    call = pl.pallas_call(
        _scores_body,
        grid=(B, nj),
        in_specs=[
            pl.BlockSpec((1, TT, D), lambda i, j: (i, j, 0)),
            pl.BlockSpec((1, D), lambda i, j: (0, 0)),
            pl.BlockSpec(memory_space=pltpu.SMEM),
        ],
        out_specs=pl.BlockSpec((1, 1, TT), lambda i, j: (i * nj + j, 0, 0)),
        out_shape=jax.ShapeDtypeStruct((B * nj, 1, TT), jnp.float32),
    )
    return call(x, W, b).reshape(B, t1)


def _round_bf16(v):
    u = lax.bitcast_convert_type(v, jnp.uint32)
    r = (u + jnp.uint32(0x7FFF) + ((u >> jnp.uint32(16)) & jnp.uint32(1)))
    r = r & jnp.uint32(0xFFFF0000)
    return lax.bitcast_convert_type(r, jnp.float32)


def _lane_sum(v, iota):
    for sh in (8, 4, 2, 1):
        v = v + jnp.take_along_axis(
            v, iota ^ sh, axis=0, mode=lax.GatherScatterMode.PROMISE_IN_BOUNDS)
    return v


def _sc_scores_body(t1, tsc, n_rows, t_full, d,
                    xf_hbm, w_hbm, bvec_hbm, out_hbm,
                    wb_v, buf0, buf1, out_v, bv_v, sem0, sem1):
    cid = lax.axis_index("c")
    sid = lax.axis_index("s")
    wid = sid * 2 + cid
    wpr = 32 // n_rows
    tpw = tsc // wpr
    grp = 8
    ng = tpw // grp
    n_chunks = d // _LANES

    b_row = wid // wpr
    part = wid % wpr
    flat_base = b_row * t_full + t1 + part * tpw

    pltpu.sync_copy(w_hbm.at[0], wb_v)
    pltpu.sync_copy(bvec_hbm, bv_v)

    def _round_w(j, carry):
        wv = wb_v[pl.ds(j * _LANES, _LANES)]
        wb_v[pl.ds(j * _LANES, _LANES)] = _round_bf16(wv)
        return carry
    lax.fori_loop(0, n_chunks, _round_w, 0, unroll=4)
    bias = bv_v[pl.ds(0, _LANES)]
    iota = lax.iota(jnp.int32, _LANES)

    bufs = (buf0, buf1)
    sems = (sem0, sem1)

    def _start(g):
        cp = pltpu.make_async_copy(
            xf_hbm.at[pl.ds(flat_base + g * grp, grp)], bufs[g % 2], sems[g % 2])
        cp.start()
        return cp

    pend = _start(0)
    for g in range(ng):
        nxt = _start(g + 1) if g + 1 < ng else None
        pend.wait()
        buf = bufs[g % 2]
        gv = jnp.zeros((_LANES,), jnp.float32)
        for tk in range(grp):
            def _dot(j, acc):
                xv = buf[tk, pl.ds(j * _LANES, _LANES)]
                xb = _round_bf16(xv)
                return acc + xb * wb_v[pl.ds(j * _LANES, _LANES)]
            acc = lax.fori_loop(0, n_chunks, _dot,
                                jnp.zeros((_LANES,), jnp.float32), unroll=8)
            tot = _lane_sum(acc, iota) + bias
            gv = jnp.where(iota == tk, tot, gv)
        plsc.store_scatter(out_v, [g * grp + iota], gv, mask=iota < grp)
        pend = nxt

    pltpu.sync_copy(out_v, out_hbm.at[b_row, pl.ds(part * tpw, tpw)])


def _sc_scores(x, W, b, t1):
    B, T, D = x.shape
    tsc = T - t1
    mesh = plsc.VectorSubcoreMesh(core_axis_name="c", subcore_axis_name="s")
    wpr = 32 // B
    tpw = tsc // wpr
    fn = pl.kernel(
        functools.partial(_sc_scores_body, t1, tsc, B, T, D),
        out_type=jax.ShapeDtypeStruct((B, tsc), jnp.float32),
        mesh=mesh,
        compiler_params=pltpu.CompilerParams(needs_layout_passes=False),
        scratch_types=[
            pltpu.VMEM((D,), jnp.float32),
            pltpu.VMEM((8, D), jnp.float32),
            pltpu.VMEM((8, D), jnp.float32),
            pltpu.VMEM((tpw,), jnp.float32),
            pltpu.VMEM((_LANES,), jnp.float32),
            pltpu.SemaphoreType.DMA,
            pltpu.SemaphoreType.DMA,
        ],
    )
    xf = x.reshape(B * T, D)
    bvec = jnp.broadcast_to(b, (_LANES,))
    return fn(xf, W, bvec)


def _select_body(k_count, n_rows, t_len, scores_hbm, mask_hbm, aux_hbm,
                 row_v, key_v, mask_v, sig_v, comb_v, aux_v, shm):
    cid = lax.axis_index("c")
    sid = lax.axis_index("s")
    n_chunks = t_len // _LANES

    @pl.when(jnp.logical_and(cid == 0, sid < n_rows))
    def _row_work():
        row = sid
        pltpu.sync_copy(scores_hbm.at[row], row_v)

        def _build(j, carry):
            s = row_v[pl.ds(j * _LANES, _LANES)]
            bits = lax.bitcast_convert_type(s, jnp.uint32)
            neg = (bits >> jnp.uint32(31)) != jnp.uint32(0)
            key = jnp.where(neg, ~bits, bits | jnp.uint32(0x80000000))
            key_v[pl.ds(j * _LANES, _LANES)] = key
            return carry
        lax.fori_loop(0, n_chunks, _build, 0, unroll=4)

        def _bit_step(i, prefix):
            bit = jnp.uint32(31) - i.astype(jnp.uint32)
            cand = prefix | (jnp.uint32(1) << bit)

            def _cnk(j, acc):
                key = key_v[pl.ds(j * _LANES, _LANES)]
                return acc + plsc.all_reduce_population_count(key >= cand)
            cnt = lax.fori_loop(0, n_chunks, _cnk,
                                jnp.zeros((_LANES,), jnp.int32), unroll=4)
            return jnp.where(cnt >= k_count, cand, prefix)
        kth = lax.fori_loop(0, 32, _bit_step,
                            jnp.zeros((_LANES,), jnp.uint32))

        def _cgt(j, acc):
            key = key_v[pl.ds(j * _LANES, _LANES)]
            return acc + plsc.all_reduce_population_count(key > kth)
        cnt_gt = lax.fori_loop(0, n_chunks, _cgt,
                               jnp.zeros((_LANES,), jnp.int32), unroll=4)
        need = k_count - cnt_gt

        iota = lax.iota(jnp.int32, _LANES)

        def _idx_step(i, ib):
            cand = ib | (jnp.int32(1) << (jnp.int32(12) - i))

            def _cnk(j, acc):
                key = key_v[pl.ds(j * _LANES, _LANES)]
                idx = j * _LANES + iota
                m = jnp.logical_and(key == kth, idx < cand)
                return acc + plsc.all_reduce_population_count(m)
            cnt = lax.fori_loop(0, n_chunks, _cnk,
                                jnp.zeros((_LANES,), jnp.int32), unroll=4)
            return jnp.where(cnt < need, cand, ib)
        ibound = lax.fori_loop(0, 13, _idx_step,
                               jnp.zeros((_LANES,), jnp.int32))

        def _write(j, carry):
            key = key_v[pl.ds(j * _LANES, _LANES)]
            s = row_v[pl.ds(j * _LANES, _LANES)]
            idx = j * _LANES + iota
            m = (key > kth) | jnp.logical_and(key == kth, idx <= ibound)
            mask_v[pl.ds(j * _LANES, _LANES)] = jnp.where(m, 1.0, 0.0).astype(jnp.float32)
            sig_v[pl.ds(j * _LANES, _LANES)] = 1.0 / (1.0 + jnp.exp(-s))
            return carry
        lax.fori_loop(0, n_chunks, _write, 0, unroll=2)

        pltpu.sync_copy(mask_v, mask_hbm.at[row])
        pltpu.sync_copy(mask_v, shm.at[row])
        pltpu.sync_copy(sig_v, shm.at[n_rows + row])

    plsc.subcore_barrier()

    @pl.when(jnp.logical_and(cid == 0, sid == 0))
    def _aux_work():
        pltpu.sync_copy(shm, comb_v)

        def _aux(j, acc):
            sl = pl.ds(j * _LANES, _LANES)
            msum = (comb_v[0, sl] + comb_v[1, sl]) + (comb_v[2, sl] + comb_v[3, sl])
            ssum = (comb_v[4, sl] + comb_v[5, sl]) + (comb_v[6, sl] + comb_v[7, sl])
            return acc + msum * ssum
        acc = lax.fori_loop(0, n_chunks, _aux,
                            jnp.zeros((_LANES,), jnp.float32), unroll=4)
        scale = 1.0 / (float(n_rows) * float(n_rows) * float(t_len))
        aux_v[pl.ds(0, _LANES)] = acc * scale
        pltpu.sync_copy(aux_v, aux_hbm)


def _select_topk(scores, k_count):
    B, T = scores.shape
    mesh = plsc.VectorSubcoreMesh(core_axis_name="c", subcore_axis_name="s")
    fn = pl.kernel(
        functools.partial(_select_body, k_count, B, T),
        out_type=(
            jax.ShapeDtypeStruct((B, T), jnp.float32),
            jax.ShapeDtypeStruct((_LANES,), jnp.float32),
        ),
        mesh=mesh,
        compiler_params=pltpu.CompilerParams(needs_layout_passes=False),
        scratch_types=[
            pltpu.VMEM((T,), jnp.float32),
            pltpu.VMEM((T,), jnp.uint32),
            pltpu.VMEM((T,), jnp.float32),
            pltpu.VMEM((T,), jnp.float32),
            pltpu.VMEM((2 * B, T), jnp.float32),
            pltpu.VMEM((_LANES,), jnp.float32),
            pltpu.VMEM_SHARED((2 * B, T), jnp.float32),
        ],
    )
    return fn(scores)


def kernel(x, W, b):
    B, T, D = x.shape
    k_count = max(1, int(T * CAP))
    t1 = T - T // 4
    scores_tc = _compute_scores(x, W, b, t1)
    scores_sc = _sc_scores(x, W, b, t1)
    scores = jnp.concatenate([scores_tc, scores_sc], axis=1)
    mask, aux = _select_topk(scores, k_count)
    return mask, jnp.sum(aux)


# speedup vs baseline: 1.0037x; 1.0037x over previous
import functools

import jax
import jax.numpy as jnp
from jax import lax
from jax.experimental import pallas as pl
from jax.experimental.pallas import tpu as pltpu
from jax.experimental.pallas import tpu_sc as plsc

CAP = 0.25
_LANES = 16


def _scores_body(x_ref, w_ref, b_ref, o_ref):
    xb = x_ref[0].astype(jnp.bfloat16).astype(jnp.float32)
    wb = w_ref[...].astype(jnp.bfloat16).astype(jnp.float32)
    tt = xb.shape[0]
    o_ref[...] = (jnp.sum(xb * wb, axis=-1) + b_ref[0]).reshape(1, 1, tt)


def _compute_scores(x, W, b, t1):
    B, T, D = x.shape
    TT = 1024
    nj = t1 // TT
    call = pl.pallas_call(
        _scores_body,
        grid=(B, nj),
        in_specs=[
            pl.BlockSpec((1, TT, D), lambda i, j: (i, j, 0)),
            pl.BlockSpec((1, D), lambda i, j: (0, 0)),
            pl.BlockSpec(memory_space=pltpu.SMEM),
        ],
        out_specs=pl.BlockSpec((1, 1, TT), lambda i, j: (i * nj + j, 0, 0)),
        out_shape=jax.ShapeDtypeStruct((B * nj, 1, TT), jnp.float32),
    )
    return call(x, W, b).reshape(B, t1)


def _round_bf16(v):
    u = lax.bitcast_convert_type(v, jnp.uint32)
    r = (u + jnp.uint32(0x7FFF) + ((u >> jnp.uint32(16)) & jnp.uint32(1)))
    r = r & jnp.uint32(0xFFFF0000)
    return lax.bitcast_convert_type(r, jnp.float32)


def _lane_sum(v, iota):
    for sh in (8, 4, 2, 1):
        v = v + jnp.take_along_axis(
            v, iota ^ sh, axis=0, mode=lax.GatherScatterMode.PROMISE_IN_BOUNDS)
    return v


def _sc_scores_body(t1, tsc, n_rows, t_full, d,
                    xf_hbm, w_hbm, bvec_hbm, out_hbm,
                    wb_v, buf0, buf1, out_v, bv_v, sem0, sem1):
    cid = lax.axis_index("c")
    sid = lax.axis_index("s")
    wid = sid * 2 + cid
    wpr = 32 // n_rows
    tpw = tsc // wpr
    grp = 8
    ng = tpw // grp
    n_chunks = d // _LANES

    b_row = wid // wpr
    part = wid % wpr
    flat_base = b_row * t_full + t1 + part * tpw

    pltpu.sync_copy(w_hbm.at[0], wb_v)
    pltpu.sync_copy(bvec_hbm, bv_v)

    def _round_w(j, carry):
        wv = wb_v[pl.ds(j * _LANES, _LANES)]
        wb_v[pl.ds(j * _LANES, _LANES)] = _round_bf16(wv)
        return carry
    lax.fori_loop(0, n_chunks, _round_w, 0, unroll=4)
    bias = bv_v[pl.ds(0, _LANES)]
    iota = lax.iota(jnp.int32, _LANES)

    bufs = (buf0, buf1)
    sems = (sem0, sem1)

    pltpu.make_async_copy(
        xf_hbm.at[pl.ds(flat_base, grp)], buf0, sem0).start()
    pltpu.make_async_copy(
        xf_hbm.at[pl.ds(flat_base + grp, grp)], buf1, sem1).start()

    def _pair(gp, carry):
        for half in range(2):
            buf, sem = bufs[half], sems[half]
            g = 2 * gp + half
            pltpu.make_async_copy(
                xf_hbm.at[pl.ds(flat_base, grp)], buf, sem).wait()
            gv = jnp.zeros((_LANES,), jnp.float32)
            for tk in range(grp):
                def _dot(j, acc):
                    sl0 = pl.ds(j * 2 * _LANES, _LANES)
                    sl1 = pl.ds(j * 2 * _LANES + _LANES, _LANES)
                    xb0 = _round_bf16(buf[tk, sl0])
                    xb1 = _round_bf16(buf[tk, sl1])
                    return acc + xb0 * wb_v[sl0] + xb1 * wb_v[sl1]
                acc = lax.fori_loop(0, n_chunks // 2, _dot,
                                    jnp.zeros((_LANES,), jnp.float32), unroll=4)
                tot = _lane_sum(acc, iota) + bias
                gv = jnp.where(iota == tk, tot, gv)
            plsc.store_scatter(out_v, [g * grp + iota], gv, mask=iota < grp)

            @pl.when(g + 2 < ng)
            def _():
                pltpu.make_async_copy(
                    xf_hbm.at[pl.ds(flat_base + (g + 2) * grp, grp)],
                    buf, sem).start()
        return carry
    lax.fori_loop(0, ng // 2, _pair, 0)

    pltpu.sync_copy(out_v, out_hbm.at[b_row, pl.ds(part * tpw, tpw)])


def _sc_scores(x, W, b, t1):
    B, T, D = x.shape
    tsc = T - t1
    mesh = plsc.VectorSubcoreMesh(core_axis_name="c", subcore_axis_name="s")
    wpr = 32 // B
    tpw = tsc // wpr
    fn = pl.kernel(
        functools.partial(_sc_scores_body, t1, tsc, B, T, D),
        out_type=jax.ShapeDtypeStruct((B, tsc), jnp.float32),
        mesh=mesh,
        compiler_params=pltpu.CompilerParams(needs_layout_passes=False),
        scratch_types=[
            pltpu.VMEM((D,), jnp.float32),
            pltpu.VMEM((8, D), jnp.float32),
            pltpu.VMEM((8, D), jnp.float32),
            pltpu.VMEM((tpw,), jnp.float32),
            pltpu.VMEM((_LANES,), jnp.float32),
            pltpu.SemaphoreType.DMA,
            pltpu.SemaphoreType.DMA,
        ],
    )
    xf = x.reshape(B * T, D)
    bvec = jnp.broadcast_to(b, (_LANES,))
    return fn(xf, W, bvec)


def _select_body(k_count, n_rows, t_len, scores_hbm, mask_hbm, aux_hbm,
                 row_v, key_v, mask_v, sig_v, comb_v, aux_v, shm):
    cid = lax.axis_index("c")
    sid = lax.axis_index("s")
    n_chunks = t_len // _LANES

    @pl.when(jnp.logical_and(cid == 0, sid < n_rows))
    def _row_work():
        row = sid
        pltpu.sync_copy(scores_hbm.at[row], row_v)

        def _build(j, carry):
            s = row_v[pl.ds(j * _LANES, _LANES)]
            bits = lax.bitcast_convert_type(s, jnp.uint32)
            neg = (bits >> jnp.uint32(31)) != jnp.uint32(0)
            key = jnp.where(neg, ~bits, bits | jnp.uint32(0x80000000))
            key_v[pl.ds(j * _LANES, _LANES)] = key
            return carry
        lax.fori_loop(0, n_chunks, _build, 0, unroll=4)

        def _bit_step(i, prefix):
            bit = jnp.uint32(31) - i.astype(jnp.uint32)
            cand = prefix | (jnp.uint32(1) << bit)

            def _cnk(j, acc):
                key = key_v[pl.ds(j * _LANES, _LANES)]
                return acc + plsc.all_reduce_population_count(key >= cand)
            cnt = lax.fori_loop(0, n_chunks, _cnk,
                                jnp.zeros((_LANES,), jnp.int32), unroll=4)
            return jnp.where(cnt >= k_count, cand, prefix)
        kth = lax.fori_loop(0, 32, _bit_step,
                            jnp.zeros((_LANES,), jnp.uint32))

        def _cgt(j, acc):
            key = key_v[pl.ds(j * _LANES, _LANES)]
            return acc + plsc.all_reduce_population_count(key > kth)
        cnt_gt = lax.fori_loop(0, n_chunks, _cgt,
                               jnp.zeros((_LANES,), jnp.int32), unroll=4)
        need = k_count - cnt_gt

        iota = lax.iota(jnp.int32, _LANES)

        def _idx_step(i, ib):
            cand = ib | (jnp.int32(1) << (jnp.int32(12) - i))

            def _cnk(j, acc):
                key = key_v[pl.ds(j * _LANES, _LANES)]
                idx = j * _LANES + iota
                m = jnp.logical_and(key == kth, idx < cand)
                return acc + plsc.all_reduce_population_count(m)
            cnt = lax.fori_loop(0, n_chunks, _cnk,
                                jnp.zeros((_LANES,), jnp.int32), unroll=4)
            return jnp.where(cnt < need, cand, ib)
        ibound = lax.fori_loop(0, 13, _idx_step,
                               jnp.zeros((_LANES,), jnp.int32))

        def _write(j, carry):
            key = key_v[pl.ds(j * _LANES, _LANES)]
            s = row_v[pl.ds(j * _LANES, _LANES)]
            idx = j * _LANES + iota
            m = (key > kth) | jnp.logical_and(key == kth, idx <= ibound)
            mask_v[pl.ds(j * _LANES, _LANES)] = jnp.where(m, 1.0, 0.0).astype(jnp.float32)
            sig_v[pl.ds(j * _LANES, _LANES)] = 1.0 / (1.0 + jnp.exp(-s))
            return carry
        lax.fori_loop(0, n_chunks, _write, 0, unroll=2)

        pltpu.sync_copy(mask_v, mask_hbm.at[row])
        pltpu.sync_copy(mask_v, shm.at[row])
        pltpu.sync_copy(sig_v, shm.at[n_rows + row])

    plsc.subcore_barrier()

    @pl.when(jnp.logical_and(cid == 0, sid == 0))
    def _aux_work():
        pltpu.sync_copy(shm, comb_v)

        def _aux(j, acc):
            sl = pl.ds(j * _LANES, _LANES)
            msum = (comb_v[0, sl] + comb_v[1, sl]) + (comb_v[2, sl] + comb_v[3, sl])
            ssum = (comb_v[4, sl] + comb_v[5, sl]) + (comb_v[6, sl] + comb_v[7, sl])
            return acc + msum * ssum
        acc = lax.fori_loop(0, n_chunks, _aux,
                            jnp.zeros((_LANES,), jnp.float32), unroll=4)
        scale = 1.0 / (float(n_rows) * float(n_rows) * float(t_len))
        aux_v[pl.ds(0, _LANES)] = acc * scale
        pltpu.sync_copy(aux_v, aux_hbm)


def _select_topk(scores, k_count):
    B, T = scores.shape
    mesh = plsc.VectorSubcoreMesh(core_axis_name="c", subcore_axis_name="s")
    fn = pl.kernel(
        functools.partial(_select_body, k_count, B, T),
        out_type=(
            jax.ShapeDtypeStruct((B, T), jnp.float32),
            jax.ShapeDtypeStruct((_LANES,), jnp.float32),
        ),
        mesh=mesh,
        compiler_params=pltpu.CompilerParams(needs_layout_passes=False),
        scratch_types=[
            pltpu.VMEM((T,), jnp.float32),
            pltpu.VMEM((T,), jnp.uint32),
            pltpu.VMEM((T,), jnp.float32),
            pltpu.VMEM((T,), jnp.float32),
            pltpu.VMEM((2 * B, T), jnp.float32),
            pltpu.VMEM((_LANES,), jnp.float32),
            pltpu.VMEM_SHARED((2 * B, T), jnp.float32),
        ],
    )
    return fn(scores)


def kernel(x, W, b):
    B, T, D = x.shape
    k_count = max(1, int(T * CAP))
    t1 = T - T // 4
    scores_tc = _compute_scores(x, W, b, t1)
    scores_sc = _sc_scores(x, W, b, t1)
    scores = jnp.concatenate([scores_tc, scores_sc], axis=1)
    mask, aux = _select_topk(scores, k_count)
    return mask, jnp.sum(aux)


# speedup vs baseline: 1.0606x; 1.0567x over previous
import functools

import jax
import jax.numpy as jnp
from jax import lax
from jax.experimental import pallas as pl
from jax.experimental.pallas import tpu as pltpu
from jax.experimental.pallas import tpu_sc as plsc

CAP = 0.25
_LANES = 16


def _scores_body(x_ref, w_ref, b_ref, o_ref):
    xb = x_ref[0].astype(jnp.bfloat16).astype(jnp.float32)
    wb = w_ref[...].astype(jnp.bfloat16).astype(jnp.float32)
    tt = xb.shape[0]
    o_ref[...] = (jnp.sum(xb * wb, axis=-1) + b_ref[0]).reshape(1, 1, tt)


def _compute_scores(x, W, b, t1):
    B, T, D = x.shape
    TT = 1024
    nj = t1 // TT
    call = pl.pallas_call(
        _scores_body,
        grid=(B, nj),
        in_specs=[
            pl.BlockSpec((1, TT, D), lambda i, j: (i, j, 0)),
            pl.BlockSpec((1, D), lambda i, j: (0, 0)),
            pl.BlockSpec(memory_space=pltpu.SMEM),
        ],
        out_specs=pl.BlockSpec((1, 1, TT), lambda i, j: (i * nj + j, 0, 0)),
        out_shape=jax.ShapeDtypeStruct((B * nj, 1, TT), jnp.float32),
    )
    return call(x, W, b).reshape(B, t1)


def _round_bf16(v):
    u = lax.bitcast_convert_type(v, jnp.uint32)
    r = (u + jnp.uint32(0x7FFF) + ((u >> jnp.uint32(16)) & jnp.uint32(1)))
    r = r & jnp.uint32(0xFFFF0000)
    return lax.bitcast_convert_type(r, jnp.float32)


def _lane_sum(v, iota):
    for sh in (8, 4, 2, 1):
        v = v + jnp.take_along_axis(
            v, iota ^ sh, axis=0, mode=lax.GatherScatterMode.PROMISE_IN_BOUNDS)
    return v


def _sc_scores_body(t1, tsc, n_rows, t_full, d,
                    xf_hbm, w_hbm, bvec_hbm, out_hbm,
                    wb_v, buf0, buf1, out_v, bv_v, sem0, sem1):
    cid = lax.axis_index("c")
    sid = lax.axis_index("s")
    wid = sid * 2 + cid
    wpr = 32 // n_rows
    tpw = tsc // wpr
    grp = 8
    ng = tpw // grp
    n_chunks = d // _LANES

    b_row = wid // wpr
    part = wid % wpr
    flat_base = b_row * t_full + t1 + part * tpw

    pltpu.sync_copy(w_hbm.at[0], wb_v)
    pltpu.sync_copy(bvec_hbm, bv_v)

    def _round_w(j, carry):
        wv = wb_v[pl.ds(j * _LANES, _LANES)]
        wb_v[pl.ds(j * _LANES, _LANES)] = _round_bf16(wv)
        return carry
    lax.fori_loop(0, n_chunks, _round_w, 0, unroll=4)
    bias = bv_v[pl.ds(0, _LANES)]
    iota = lax.iota(jnp.int32, _LANES)

    bufs = (buf0, buf1)
    sems = (sem0, sem1)

    pltpu.make_async_copy(
        xf_hbm.at[pl.ds(flat_base, grp)], buf0, sem0).start()
    pltpu.make_async_copy(
        xf_hbm.at[pl.ds(flat_base + grp, grp)], buf1, sem1).start()

    def _pair(gp, carry):
        for half in range(2):
            buf, sem = bufs[half], sems[half]
            g = 2 * gp + half
            pltpu.make_async_copy(
                xf_hbm.at[pl.ds(flat_base, grp)], buf, sem).wait()
            gv = jnp.zeros((_LANES,), jnp.float32)
            for tk in range(grp):
                def _dot(j, acc):
                    sl0 = pl.ds(j * 2 * _LANES, _LANES)
                    sl1 = pl.ds(j * 2 * _LANES + _LANES, _LANES)
                    xb0 = _round_bf16(buf[tk, sl0])
                    xb1 = _round_bf16(buf[tk, sl1])
                    return acc + xb0 * wb_v[sl0] + xb1 * wb_v[sl1]
                acc = lax.fori_loop(0, n_chunks // 2, _dot,
                                    jnp.zeros((_LANES,), jnp.float32), unroll=4)
                tot = _lane_sum(acc, iota) + bias
                gv = jnp.where(iota == tk, tot, gv)
            plsc.store_scatter(out_v, [g * grp + iota], gv, mask=iota < grp)

            @pl.when(g + 2 < ng)
            def _():
                pltpu.make_async_copy(
                    xf_hbm.at[pl.ds(flat_base + (g + 2) * grp, grp)],
                    buf, sem).start()
        return carry
    lax.fori_loop(0, ng // 2, _pair, 0)

    pltpu.sync_copy(out_v, out_hbm.at[b_row, pl.ds(part * tpw, tpw)])


def _sc_scores(x, W, b, t1):
    B, T, D = x.shape
    tsc = T - t1
    mesh = plsc.VectorSubcoreMesh(core_axis_name="c", subcore_axis_name="s")
    wpr = 32 // B
    tpw = tsc // wpr
    fn = pl.kernel(
        functools.partial(_sc_scores_body, t1, tsc, B, T, D),
        out_type=jax.ShapeDtypeStruct((B, tsc), jnp.float32),
        mesh=mesh,
        compiler_params=pltpu.CompilerParams(needs_layout_passes=False),
        scratch_types=[
            pltpu.VMEM((D,), jnp.float32),
            pltpu.VMEM((8, D), jnp.float32),
            pltpu.VMEM((8, D), jnp.float32),
            pltpu.VMEM((tpw,), jnp.float32),
            pltpu.VMEM((_LANES,), jnp.float32),
            pltpu.SemaphoreType.DMA,
            pltpu.SemaphoreType.DMA,
        ],
    )
    xf = x.reshape(B * T, D)
    bvec = jnp.broadcast_to(b, (_LANES,))
    return fn(xf, W, bvec)


def _select_body(k_count, n_rows, t_len, scores_hbm, mask_hbm, aux_hbm,
                 row_v, key_v, mask_v, sig_v, comb_v, aux_v, ib_v, shm):
    cid = lax.axis_index("c")
    sid = lax.axis_index("s")
    n_chunks = t_len // _LANES

    @pl.when(jnp.logical_and(cid == 0, sid < n_rows))
    def _row_work():
        row = sid
        pltpu.sync_copy(scores_hbm.at[row], row_v)

        def _build(j, carry):
            s = row_v[pl.ds(j * _LANES, _LANES)]
            bits = lax.bitcast_convert_type(s, jnp.uint32)
            neg = (bits >> jnp.uint32(31)) != jnp.uint32(0)
            key = jnp.where(neg, ~bits, bits | jnp.uint32(0x80000000))
            key_v[pl.ds(j * _LANES, _LANES)] = key
            return carry
        lax.fori_loop(0, n_chunks, _build, 0, unroll=4)

        def _bit_step(i, prefix):
            bit = jnp.uint32(31) - i.astype(jnp.uint32)
            cand = prefix | (jnp.uint32(1) << bit)

            def _cnk(j, acc):
                key = key_v[pl.ds(j * _LANES, _LANES)]
                return acc + plsc.all_reduce_population_count(key >= cand)
            cnt = lax.fori_loop(0, n_chunks, _cnk,
                                jnp.zeros((_LANES,), jnp.int32), unroll=8)
            return jnp.where(cnt >= k_count, cand, prefix)
        kth = lax.fori_loop(0, 32, _bit_step,
                            jnp.zeros((_LANES,), jnp.uint32))

        def _cge(j, accs):
            key = key_v[pl.ds(j * _LANES, _LANES)]
            return (accs[0] + plsc.all_reduce_population_count(key > kth),
                    accs[1] + plsc.all_reduce_population_count(key >= kth))
        cnt_gt, cnt_ge = lax.fori_loop(
            0, n_chunks, _cge,
            (jnp.zeros((_LANES,), jnp.int32), jnp.zeros((_LANES,), jnp.int32)),
            unroll=8)
        need = k_count - cnt_gt

        iota = lax.iota(jnp.int32, _LANES)
        ib_v[pl.ds(0, _LANES)] = jnp.full((_LANES,), t_len, jnp.int32)

        @pl.when(cnt_ge[0] != k_count)
        def _break_ties():
            def _idx_step(i, ib):
                cand = ib | (jnp.int32(1) << (jnp.int32(12) - i))

                def _cnk(j, acc):
                    key = key_v[pl.ds(j * _LANES, _LANES)]
                    idx = j * _LANES + iota
                    m = jnp.logical_and(key == kth, idx < cand)
                    return acc + plsc.all_reduce_population_count(m)
                cnt = lax.fori_loop(0, n_chunks, _cnk,
                                    jnp.zeros((_LANES,), jnp.int32), unroll=4)
                return jnp.where(cnt < need, cand, ib)
            ib_v[pl.ds(0, _LANES)] = lax.fori_loop(
                0, 13, _idx_step, jnp.zeros((_LANES,), jnp.int32))

        ibound = ib_v[pl.ds(0, _LANES)]

        def _write(j, carry):
            key = key_v[pl.ds(j * _LANES, _LANES)]
            s = row_v[pl.ds(j * _LANES, _LANES)]
            idx = j * _LANES + iota
            m = (key > kth) | jnp.logical_and(key == kth, idx <= ibound)
            mask_v[pl.ds(j * _LANES, _LANES)] = jnp.where(m, 1.0, 0.0).astype(jnp.float32)
            sig_v[pl.ds(j * _LANES, _LANES)] = 1.0 / (1.0 + jnp.exp(-s))
            return carry
        lax.fori_loop(0, n_chunks, _write, 0, unroll=2)

        pltpu.sync_copy(mask_v, mask_hbm.at[row])
        pltpu.sync_copy(mask_v, shm.at[row])
        pltpu.sync_copy(sig_v, shm.at[n_rows + row])

    plsc.subcore_barrier()

    @pl.when(jnp.logical_and(cid == 0, sid == 0))
    def _aux_work():
        pltpu.sync_copy(shm, comb_v)

        def _aux(j, acc):
            sl = pl.ds(j * _LANES, _LANES)
            msum = (comb_v[0, sl] + comb_v[1, sl]) + (comb_v[2, sl] + comb_v[3, sl])
            ssum = (comb_v[4, sl] + comb_v[5, sl]) + (comb_v[6, sl] + comb_v[7, sl])
            return acc + msum * ssum
        acc = lax.fori_loop(0, n_chunks, _aux,
                            jnp.zeros((_LANES,), jnp.float32), unroll=4)
        scale = 1.0 / (float(n_rows) * float(n_rows) * float(t_len))
        aux_v[pl.ds(0, _LANES)] = acc * scale
        pltpu.sync_copy(aux_v, aux_hbm)


def _select_topk(scores, k_count):
    B, T = scores.shape
    mesh = plsc.VectorSubcoreMesh(core_axis_name="c", subcore_axis_name="s")
    fn = pl.kernel(
        functools.partial(_select_body, k_count, B, T),
        out_type=(
            jax.ShapeDtypeStruct((B, T), jnp.float32),
            jax.ShapeDtypeStruct((_LANES,), jnp.float32),
        ),
        mesh=mesh,
        compiler_params=pltpu.CompilerParams(needs_layout_passes=False),
        scratch_types=[
            pltpu.VMEM((T,), jnp.float32),
            pltpu.VMEM((T,), jnp.uint32),
            pltpu.VMEM((T,), jnp.float32),
            pltpu.VMEM((T,), jnp.float32),
            pltpu.VMEM((2 * B, T), jnp.float32),
            pltpu.VMEM((_LANES,), jnp.float32),
            pltpu.VMEM((_LANES,), jnp.int32),
            pltpu.VMEM_SHARED((2 * B, T), jnp.float32),
        ],
    )
    return fn(scores)


def kernel(x, W, b):
    B, T, D = x.shape
    k_count = max(1, int(T * CAP))
    t1 = T - T // 4
    scores_tc = _compute_scores(x, W, b, t1)
    scores_sc = _sc_scores(x, W, b, t1)
    scores = jnp.concatenate([scores_tc, scores_sc], axis=1)
    mask, aux = _select_topk(scores, k_count)
    return mask, jnp.sum(aux)
